# baseline (speedup 1.0000x reference)
import jax
import jax.numpy as jnp
import numpy as np
from jax import lax
from jax.experimental import pallas as pl
from jax.experimental.pallas import tpu as pltpu
from jax.experimental.pallas import tpu_sc as plsc

_B = 64
_V = 100000
_L = 16
_NC = 2
_NS = 16
_NW = _NC * _NS
_K = 16
_SW = 25088
_CW = 3584
_NCH = 7
_C6 = 3200
_UN = 10


def _sweep(buf, r8, start_vec, nblocks, unroll, m, acc):
    def blk(i, carry):
        m, acc = carry
        base = start_vec * _L + i * (_L * unroll)
        xs = [buf[r8, pl.ds(base + u * _L, _L)] for u in range(unroll)]
        bm = xs[0]
        for u in range(1, unroll):
            bm = jnp.maximum(bm, xs[u])
        nm = jnp.maximum(m, bm)
        acc = acc * jnp.exp(m - nm)
        for u in range(unroll):
            acc = acc + jnp.exp(xs[u] - nm)
        return nm, acc
    return lax.fori_loop(0, nblocks, blk, (m, acc))


def _sc_body(logits_hbm, tail_hbm, cr8_hbm, cgc_hbm, cnz_hbm, out_hbm,
             bufa, bufb, tbuf, cr8, cgc, cnz, xcs, stg, rbuf, ribuf, gbuf,
             ibuf, vbuf, shm, shb, shi, sem0, sem1, semp):
    cid = lax.axis_index("c")
    sid = lax.axis_index("s")
    rg = cid * 4 + sid // 4
    cg = sid % 4
    widx = cid * _NS + sid
    iota = lax.iota(jnp.int32, _L)
    r0 = pl.multiple_of(rg * 8, 8)
    sbase = cg * _SW
    bufs = (bufa, bufb)
    sems = (sem0, sem1)
    ninf = jnp.float32(-1e38)
    imax = jnp.int32(2**31 - 1)

    pltpu.sync_copy(cr8_hbm.at[widx], cr8)
    pltpu.sync_copy(cgc_hbm.at[widx], cgc)
    pltpu.sync_copy(cnz_hbm.at[widx], cnz)
    pltpu.sync_copy(tail_hbm.at[pl.ds(r0, 8)], tbuf)

    def chunk_dma(c, slot):
        cb = pl.multiple_of(sbase + c * _CW, 128)
        return pltpu.async_copy(
            logits_hbm.at[pl.ds(r0, 8), pl.ds(cb, _CW)], bufs[slot],
            sems[slot])

    def chunk6_common_dma(slot):
        cb = pl.multiple_of(sbase + 6 * _CW, 128)
        return pltpu.async_copy(
            logits_hbm.at[pl.ds(r0, 8), pl.ds(cb, _C6)],
            bufs[slot].at[:, pl.ds(0, _C6)], sems[slot])

    def chunk6_tail_dmas(slot):
        cb = pl.multiple_of(sbase + 6 * _CW + _C6, 128)
        tails = []
        @pl.when(cg < 3)
        def _():
            tails.append(pltpu.async_copy(
                logits_hbm.at[pl.ds(r0, 8), pl.ds(cb, 384)],
                bufs[slot].at[:, pl.ds(_C6, 384)], semp))
        return tails

    state = {r8: (jnp.full((_L,), ninf, jnp.float32),
                  jnp.zeros((_L,), jnp.float32)) for r8 in range(8)}

    copies = {0: chunk_dma(0, 0)}
    tail_copies = None
    for c in range(_NCH):
        slot = c % 2
        buf = bufs[slot]
        if c + 1 < _NCH - 1:
            copies[c + 1] = chunk_dma(c + 1, (c + 1) % 2)
        elif c + 1 == _NCH - 1:
            copies[c + 1] = chunk6_common_dma((c + 1) % 2)
            tail_copies = chunk6_tail_dmas((c + 1) % 2)
        copies.pop(c).wait()
        if c == _NCH - 1:
            @pl.when(cg < 3)
            def _():
                tail_copies[0].wait()

        if c < _NCH - 1:
            nfull = (_CW // _L) // _UN
            remf = _CW // _L - nfull * _UN
            for r8 in range(8):
                st = _sweep(buf, r8, 0, nfull, _UN, *state[r8])
                if remf:
                    st = _sweep(buf, r8, nfull * _UN, 1, remf, *st)
                state[r8] = st
            cwidth = _CW
            cb_t = sbase + c * _CW
        else:
            npost = jnp.where(cg == 3, 0, 12)
            ntail = jnp.where(cg == 3, 1, 0)
            for r8 in range(8):
                st = _sweep(buf, r8, 0, _C6 // (_L * _UN), _UN, *state[r8])
                rem = _C6 // _L - (_C6 // (_L * _UN)) * _UN
                if rem:
                    st = _sweep(buf, r8, _C6 // _L - rem, 1, rem, *st)
                st = _sweep(buf, r8, _C6 // _L, npost, 2, *st)
                state[r8] = _sweep(tbuf, r8, 0, ntail, 2, *st)
            cwidth = jnp.where(cg == 3, 3232, 3584)
            cb_t = sbase + 6 * _CW

        rv = cr8[c, pl.ds(0, _L)]
        local = cgc[c, pl.ds(0, _L)] - cb_t
        mask = (local >= 0) & (local < cwidth)
        if c < _NCH - 1:
            safe = jnp.where(mask, local, 0)
            val = plsc.load_gather(buf, [rv, safe])
        else:
            sel_t = (cg == 3) & (local >= _C6)
            safe_b = jnp.where(mask & jnp.logical_not(sel_t), local, 0)
            safe_t = jnp.where(sel_t, local - _C6, 0)
            val = jnp.where(sel_t,
                            plsc.load_gather(tbuf, [rv, safe_t]),
                            plsc.load_gather(buf, [rv, safe_b]))
        xcs[c, pl.ds(0, _L)] = jnp.where(mask, val, ninf)

    pm = jnp.full((_L,), ninf, jnp.float32)
    ps = jnp.zeros((_L,), jnp.float32)
    for r8 in range(8):
        m, acc = state[r8]
        m_sc = jnp.max(m)
        s_sc = jnp.sum(acc * jnp.exp(m - m_sc))
        pm = jnp.where(iota == r8, m_sc, pm)
        ps = jnp.where(iota == r8, s_sc, ps)

    stg[pl.ds(0, _L)] = pm
    stg[pl.ds(_L, _L)] = ps
    pltpu.sync_copy(stg, shm.at[pl.ds(sid * 32, 32)])
    plsc.subcore_barrier()
    pltpu.sync_copy(shm.at[pl.ds((sid // 4) * 128, 128)], rbuf)
    pms = [rbuf[pl.ds(j * 32, _L)] for j in range(4)]
    pss = [rbuf[pl.ds(j * 32 + _L, _L)] for j in range(4)]
    gm = pms[0]
    for j in range(1, 4):
        gm = jnp.maximum(gm, pms[j])
    gs = jnp.zeros((_L,), jnp.float32)
    for j in range(4):
        gs = gs + pss[j] * jnp.exp(pms[j] - gm)
    rinv = jnp.full((_L,), 1.0, jnp.float32) / gs
    gbuf[pl.ds(0, _L)] = gm
    gbuf[pl.ds(_L, _L)] = rinv

    lbest = jnp.full((_L,), ninf, jnp.float32)
    lidx = jnp.full((_L,), imax, jnp.int32)
    for c in range(_NCH):
        rv = cr8[c, pl.ds(0, _L)]
        gc = cgc[c, pl.ds(0, _L)]
        m_c = plsc.load_gather(gbuf, [rv])
        r_c = plsc.load_gather(gbuf, [rv + _L])
        score = jnp.exp(xcs[c, pl.ds(0, _L)] - m_c) * r_c \
            + cnz[c, pl.ds(0, _L)]
        for r8 in range(8):
            sel = rv == r8
            sc_r = jnp.where(sel, score, ninf)
            gc_r = jnp.where(sel, gc, imax)
            b = jnp.max(sc_r)
            i_r = jnp.min(jnp.where(sc_r == b, gc_r, imax))
            bvec = jnp.where(iota == r8, b, ninf)
            ivec = jnp.where(iota == r8, i_r, imax)
            better = (bvec > lbest) | ((bvec == lbest) & (ivec < lidx))
            lbest = jnp.where(better, bvec, lbest)
            lidx = jnp.where(better, ivec, lidx)

    stg[pl.ds(0, _L)] = lbest
    pltpu.sync_copy(stg.at[pl.ds(0, _L)], shb.at[pl.ds(sid * _L, _L)])
    ibuf[...] = lidx
    pltpu.sync_copy(ibuf, shi.at[pl.ds(sid * _L, _L)])
    plsc.subcore_barrier()

    @pl.when(cg == 0)
    def _():
        pltpu.sync_copy(shb.at[pl.ds(sid * _L, 4 * _L)], rbuf.at[pl.ds(0, 4 * _L)])
        bs = [rbuf[pl.ds(j * _L, _L)] for j in range(4)]
        gb = bs[0]
        for j in range(1, 4):
            gb = jnp.maximum(gb, bs[j])
        pltpu.sync_copy(shi.at[pl.ds(sid * _L, 4 * _L)], ribuf)
        gi = jnp.full((_L,), imax, jnp.int32)
        for j in range(4):
            ij = ribuf[pl.ds(j * _L, _L)]
            gi = jnp.minimum(gi, jnp.where(bs[j] == gb, ij, imax))
        ibuf[...] = gi
        for r8 in range(8):
            bvec = plsc.load_gather(ibuf, [jnp.full((_L,), r8, jnp.int32)])
            vbuf[r8, pl.ds(0, _L)] = bvec
        pltpu.sync_copy(vbuf, out_hbm.at[pl.ds(r0, 8)])


_sampler = pl.kernel(
    _sc_body,
    out_type=jax.ShapeDtypeStruct((_B, _L), jnp.int32),
    mesh=plsc.VectorSubcoreMesh(
        core_axis_name="c", subcore_axis_name="s",
        num_cores=_NC, num_subcores=_NS),
    scratch_types=[
        pltpu.VMEM((8, _CW), jnp.float32),
        pltpu.VMEM((8, _CW), jnp.float32),
        pltpu.VMEM((8, 32), jnp.float32),
        pltpu.VMEM((8, 128), jnp.int32),
        pltpu.VMEM((8, 128), jnp.int32),
        pltpu.VMEM((8, 128), jnp.float32),
        pltpu.VMEM((8, _K), jnp.float32),
        pltpu.VMEM((32,), jnp.float32),
        pltpu.VMEM((128,), jnp.float32),
        pltpu.VMEM((64,), jnp.int32),
        pltpu.VMEM((32,), jnp.float32),
        pltpu.VMEM((_L,), jnp.int32),
        pltpu.VMEM((8, _L), jnp.int32),
        pltpu.VMEM_SHARED((_NS * 32,), jnp.float32),
        pltpu.VMEM_SHARED((_NS * _L,), jnp.float32),
        pltpu.VMEM_SHARED((_NS * _L,), jnp.int32),
        pltpu.SemaphoreType.DMA,
        pltpu.SemaphoreType.DMA,
        pltpu.SemaphoreType.DMA,
    ],
    compiler_params=pltpu.CompilerParams(needs_layout_passes=False),
)


_CONST_CACHE = []


def _candidates():
    if not _CONST_CACHE:
        try:
            with jax.ensure_compile_time_eval():
                u = jax.random.uniform(jax.random.key(1), (_B, _V),
                                       minval=1e-9, maxval=1.0,
                                       dtype=jnp.float32)
                n = np.asarray(-jnp.log(-jnp.log(u)))
        except Exception:
            u_np = np.random.default_rng(1).uniform(
                1e-9, 1.0, (_B, _V)).astype(np.float32)
            n = (-np.log(-np.log(u_np))).astype(np.float32)
        thresh = n.max(axis=1, keepdims=True) - np.float32(1.001)
        cr8 = np.zeros((_NW, 8, 128), np.int32)
        cgc = np.zeros((_NW, 8, 128), np.int32)
        cnz = np.full((_NW, 8, 128), -1e30, np.float32)
        fill = np.zeros((_NW, 8), np.int32)
        for b in range(_B):
            for col in np.nonzero(n[b] >= thresh[b])[0]:
                rg, r8 = b // 8, b % 8
                cg = min(int(col) // _SW, 3)
                c = min((int(col) - cg * _SW) // _CW, _NCH - 1)
                w = (rg // 4) * _NS + (rg % 4) * 4 + cg
                k = fill[w, c]
                assert k < _K, (w, c, k)
                cr8[w, c, k] = r8
                cgc[w, c, k] = col
                cnz[w, c, k] = n[b, col]
                fill[w, c] = k + 1
        _CONST_CACHE.append(
            (jnp.asarray(cr8), jnp.asarray(cgc), jnp.asarray(cnz)))
    return _CONST_CACHE[0]


def kernel(logits):
    cr8, cgc, cnz = _candidates()
    tail = lax.slice(logits, (0, _V - 32), (_B, _V))
    out = _sampler(logits, tail, cr8, cgc, cnz)
    return out[:, 0]


# speedup vs baseline: 1.0307x; 1.0307x over previous
import jax
import jax.numpy as jnp
import numpy as np
from jax import lax
from jax.experimental import pallas as pl
from jax.experimental.pallas import tpu as pltpu
from jax.experimental.pallas import tpu_sc as plsc

_B = 64
_V = 100000
_L = 16
_NC = 2
_NS = 16
_NW = _NC * _NS
_K = 16
_SW = 25088
_CW = 3584
_NCH = 7
_C6 = 3200
_UN = 8


def _sweep(buf, r8, start_vec, nblocks, unroll, m, acc):
    def blk(i, carry):
        m, acc = carry
        base = start_vec * _L + i * (_L * unroll)
        xs = [buf[r8, pl.ds(base + u * _L, _L)] for u in range(unroll)]
        bm = xs[0]
        for u in range(1, unroll):
            bm = jnp.maximum(bm, xs[u])
        nm = jnp.maximum(m, bm)
        acc = acc * jnp.exp(m - nm)
        for u in range(unroll):
            acc = acc + jnp.exp(xs[u] - nm)
        return nm, acc
    return lax.fori_loop(0, nblocks, blk, (m, acc))


def _sc_body(logits_hbm, tail_hbm, cr8_hbm, cgc_hbm, cnz_hbm, out_hbm,
             bufa, bufb, tbuf, cr8, cgc, cnz, xcs, stg, rbuf, ribuf, gbuf,
             ibuf, vbuf, shm, shb, shi, sem0, sem1, semp):
    cid = lax.axis_index("c")
    sid = lax.axis_index("s")
    rg = cid * 4 + sid // 4
    cg = sid % 4
    widx = cid * _NS + sid
    iota = lax.iota(jnp.int32, _L)
    r0 = pl.multiple_of(rg * 8, 8)
    sbase = cg * _SW
    bufs = (bufa, bufb)
    sems = (sem0, sem1)
    ninf = jnp.float32(-1e38)
    imax = jnp.int32(2**31 - 1)

    pltpu.sync_copy(cr8_hbm.at[widx], cr8)
    pltpu.sync_copy(cgc_hbm.at[widx], cgc)
    pltpu.sync_copy(cnz_hbm.at[widx], cnz)
    pltpu.sync_copy(tail_hbm.at[pl.ds(r0, 8)], tbuf)

    def chunk_dma(c, slot):
        cb = pl.multiple_of(sbase + c * _CW, 128)
        return pltpu.async_copy(
            logits_hbm.at[pl.ds(r0, 8), pl.ds(cb, _CW)], bufs[slot],
            sems[slot])

    def chunk6_common_dma(slot):
        cb = pl.multiple_of(sbase + 6 * _CW, 128)
        return pltpu.async_copy(
            logits_hbm.at[pl.ds(r0, 8), pl.ds(cb, _C6)],
            bufs[slot].at[:, pl.ds(0, _C6)], sems[slot])

    def chunk6_tail_dmas(slot):
        cb = pl.multiple_of(sbase + 6 * _CW + _C6, 128)
        tails = []
        @pl.when(cg < 3)
        def _():
            tails.append(pltpu.async_copy(
                logits_hbm.at[pl.ds(r0, 8), pl.ds(cb, 384)],
                bufs[slot].at[:, pl.ds(_C6, 384)], semp))
        return tails

    state = {r8: (jnp.full((_L,), ninf, jnp.float32),
                  jnp.zeros((_L,), jnp.float32)) for r8 in range(8)}

    copies = {0: chunk_dma(0, 0)}
    tail_copies = None
    for c in range(_NCH):
        slot = c % 2
        buf = bufs[slot]
        if c + 1 < _NCH - 1:
            copies[c + 1] = chunk_dma(c + 1, (c + 1) % 2)
        elif c + 1 == _NCH - 1:
            copies[c + 1] = chunk6_common_dma((c + 1) % 2)
            tail_copies = chunk6_tail_dmas((c + 1) % 2)
        copies.pop(c).wait()
        if c == _NCH - 1:
            @pl.when(cg < 3)
            def _():
                tail_copies[0].wait()

        if c < _NCH - 1:
            nfull = (_CW // _L) // _UN
            remf = _CW // _L - nfull * _UN
            for r8 in range(8):
                st = _sweep(buf, r8, 0, nfull, _UN, *state[r8])
                if remf:
                    st = _sweep(buf, r8, nfull * _UN, 1, remf, *st)
                state[r8] = st
            cwidth = _CW
            cb_t = sbase + c * _CW
        else:
            npost = jnp.where(cg == 3, 0, 12)
            ntail = jnp.where(cg == 3, 1, 0)
            for r8 in range(8):
                st = _sweep(buf, r8, 0, _C6 // (_L * _UN), _UN, *state[r8])
                rem = _C6 // _L - (_C6 // (_L * _UN)) * _UN
                if rem:
                    st = _sweep(buf, r8, _C6 // _L - rem, 1, rem, *st)
                st = _sweep(buf, r8, _C6 // _L, npost, 2, *st)
                state[r8] = _sweep(tbuf, r8, 0, ntail, 2, *st)
            cwidth = jnp.where(cg == 3, 3232, 3584)
            cb_t = sbase + 6 * _CW

        rv = cr8[c, pl.ds(0, _L)]
        local = cgc[c, pl.ds(0, _L)] - cb_t
        mask = (local >= 0) & (local < cwidth)
        if c < _NCH - 1:
            safe = jnp.where(mask, local, 0)
            val = plsc.load_gather(buf, [rv, safe])
        else:
            sel_t = (cg == 3) & (local >= _C6)
            safe_b = jnp.where(mask & jnp.logical_not(sel_t), local, 0)
            safe_t = jnp.where(sel_t, local - _C6, 0)
            val = jnp.where(sel_t,
                            plsc.load_gather(tbuf, [rv, safe_t]),
                            plsc.load_gather(buf, [rv, safe_b]))
        xcs[c, pl.ds(0, _L)] = jnp.where(mask, val, ninf)

    pm = jnp.full((_L,), ninf, jnp.float32)
    ps = jnp.zeros((_L,), jnp.float32)
    for r8 in range(8):
        m, acc = state[r8]
        m_sc = jnp.max(m)
        s_sc = jnp.sum(acc * jnp.exp(m - m_sc))
        pm = jnp.where(iota == r8, m_sc, pm)
        ps = jnp.where(iota == r8, s_sc, ps)

    stg[pl.ds(0, _L)] = pm
    stg[pl.ds(_L, _L)] = ps
    pltpu.sync_copy(stg, shm.at[pl.ds(sid * 32, 32)])
    plsc.subcore_barrier()
    pltpu.sync_copy(shm.at[pl.ds((sid // 4) * 128, 128)], rbuf)
    pms = [rbuf[pl.ds(j * 32, _L)] for j in range(4)]
    pss = [rbuf[pl.ds(j * 32 + _L, _L)] for j in range(4)]
    gm = pms[0]
    for j in range(1, 4):
        gm = jnp.maximum(gm, pms[j])
    gs = jnp.zeros((_L,), jnp.float32)
    for j in range(4):
        gs = gs + pss[j] * jnp.exp(pms[j] - gm)
    rinv = jnp.full((_L,), 1.0, jnp.float32) / gs
    gbuf[pl.ds(0, _L)] = gm
    gbuf[pl.ds(_L, _L)] = rinv

    lbest = jnp.full((_L,), ninf, jnp.float32)
    lidx = jnp.full((_L,), imax, jnp.int32)
    for c in range(_NCH):
        rv = cr8[c, pl.ds(0, _L)]
        gc = cgc[c, pl.ds(0, _L)]
        m_c = plsc.load_gather(gbuf, [rv])
        r_c = plsc.load_gather(gbuf, [rv + _L])
        score = jnp.exp(xcs[c, pl.ds(0, _L)] - m_c) * r_c \
            + cnz[c, pl.ds(0, _L)]
        for r8 in range(8):
            sel = rv == r8
            sc_r = jnp.where(sel, score, ninf)
            gc_r = jnp.where(sel, gc, imax)
            b = jnp.max(sc_r)
            i_r = jnp.min(jnp.where(sc_r == b, gc_r, imax))
            bvec = jnp.where(iota == r8, b, ninf)
            ivec = jnp.where(iota == r8, i_r, imax)
            better = (bvec > lbest) | ((bvec == lbest) & (ivec < lidx))
            lbest = jnp.where(better, bvec, lbest)
            lidx = jnp.where(better, ivec, lidx)

    stg[pl.ds(0, _L)] = lbest
    pltpu.sync_copy(stg.at[pl.ds(0, _L)], shb.at[pl.ds(sid * _L, _L)])
    ibuf[...] = lidx
    pltpu.sync_copy(ibuf, shi.at[pl.ds(sid * _L, _L)])
    plsc.subcore_barrier()

    @pl.when(cg == 0)
    def _():
        pltpu.sync_copy(shb.at[pl.ds(sid * _L, 4 * _L)], rbuf.at[pl.ds(0, 4 * _L)])
        bs = [rbuf[pl.ds(j * _L, _L)] for j in range(4)]
        gb = bs[0]
        for j in range(1, 4):
            gb = jnp.maximum(gb, bs[j])
        pltpu.sync_copy(shi.at[pl.ds(sid * _L, 4 * _L)], ribuf)
        gi = jnp.full((_L,), imax, jnp.int32)
        for j in range(4):
            ij = ribuf[pl.ds(j * _L, _L)]
            gi = jnp.minimum(gi, jnp.where(bs[j] == gb, ij, imax))
        ibuf[...] = gi
        for r8 in range(8):
            bvec = plsc.load_gather(ibuf, [jnp.full((_L,), r8, jnp.int32)])
            vbuf[r8, pl.ds(0, _L)] = bvec
        pltpu.sync_copy(vbuf, out_hbm.at[pl.ds(r0, 8)])


_sampler = pl.kernel(
    _sc_body,
    out_type=jax.ShapeDtypeStruct((_B, _L), jnp.int32),
    mesh=plsc.VectorSubcoreMesh(
        core_axis_name="c", subcore_axis_name="s",
        num_cores=_NC, num_subcores=_NS),
    scratch_types=[
        pltpu.VMEM((8, _CW), jnp.float32),
        pltpu.VMEM((8, _CW), jnp.float32),
        pltpu.VMEM((8, 32), jnp.float32),
        pltpu.VMEM((8, 128), jnp.int32),
        pltpu.VMEM((8, 128), jnp.int32),
        pltpu.VMEM((8, 128), jnp.float32),
        pltpu.VMEM((8, _K), jnp.float32),
        pltpu.VMEM((32,), jnp.float32),
        pltpu.VMEM((128,), jnp.float32),
        pltpu.VMEM((64,), jnp.int32),
        pltpu.VMEM((32,), jnp.float32),
        pltpu.VMEM((_L,), jnp.int32),
        pltpu.VMEM((8, _L), jnp.int32),
        pltpu.VMEM_SHARED((_NS * 32,), jnp.float32),
        pltpu.VMEM_SHARED((_NS * _L,), jnp.float32),
        pltpu.VMEM_SHARED((_NS * _L,), jnp.int32),
        pltpu.SemaphoreType.DMA,
        pltpu.SemaphoreType.DMA,
        pltpu.SemaphoreType.DMA,
    ],
    compiler_params=pltpu.CompilerParams(needs_layout_passes=False),
)


_CONST_CACHE = []


def _candidates():
    if not _CONST_CACHE:
        try:
            with jax.ensure_compile_time_eval():
                u = jax.random.uniform(jax.random.key(1), (_B, _V),
                                       minval=1e-9, maxval=1.0,
                                       dtype=jnp.float32)
                n = np.asarray(-jnp.log(-jnp.log(u)))
        except Exception:
            u_np = np.random.default_rng(1).uniform(
                1e-9, 1.0, (_B, _V)).astype(np.float32)
            n = (-np.log(-np.log(u_np))).astype(np.float32)
        thresh = n.max(axis=1, keepdims=True) - np.float32(1.001)
        cr8 = np.zeros((_NW, 8, 128), np.int32)
        cgc = np.zeros((_NW, 8, 128), np.int32)
        cnz = np.full((_NW, 8, 128), -1e30, np.float32)
        fill = np.zeros((_NW, 8), np.int32)
        for b in range(_B):
            for col in np.nonzero(n[b] >= thresh[b])[0]:
                rg, r8 = b // 8, b % 8
                cg = min(int(col) // _SW, 3)
                c = min((int(col) - cg * _SW) // _CW, _NCH - 1)
                w = (rg // 4) * _NS + (rg % 4) * 4 + cg
                k = fill[w, c]
                assert k < _K, (w, c, k)
                cr8[w, c, k] = r8
                cgc[w, c, k] = col
                cnz[w, c, k] = n[b, col]
                fill[w, c] = k + 1
        _CONST_CACHE.append(
            (jnp.asarray(cr8), jnp.asarray(cgc), jnp.asarray(cnz)))
    return _CONST_CACHE[0]


def kernel(logits):
    cr8, cgc, cnz = _candidates()
    tail = lax.slice(logits, (0, _V - 32), (_B, _V))
    out = _sampler(logits, tail, cr8, cgc, cnz)
    return out[:, 0]
